# baseline (speedup 1.0000x reference)
import functools

import jax
import jax.numpy as jnp
from jax import lax
from jax.experimental import pallas as pl
from jax.experimental.pallas import tpu as pltpu
from jax.experimental.pallas import tpu_sc as plsc


def _dot(a, b):
    return lax.dot_general(a, b, (((1,), (0,)), ((), ())),
                           preferred_element_type=jnp.float32)


def _p1_body(x1_ref, x1t_ref, x2_ref, t8_ref, i1_ref, i2_ref, i3_ref,
             *, TN, S):
    b = pl.program_id(0)

    x1b = x1_ref[0]
    x1t = x1t_ref[0]
    x2b = x2_ref[0]

    x2sq = x2b * x2b
    n2 = x2sq[0:1] + x2sq[1:2] + x2sq[2:3]
    x1tsq = x1t * x1t
    n1 = x1tsq[:, 0:1] + x1tsq[:, 1:2] + x1tsq[:, 2:3]
    prod = lax.dot_general(x1b, x2b, (((0,), (0,)), ((), ())),
                           preferred_element_type=jnp.float32)
    d = jnp.maximum(n1 + n2 - 2.0 * prod, 0.0)

    iota = lax.broadcasted_iota(jnp.int32, (TN, S), 1).astype(jnp.float32)
    inf = jnp.float32(jnp.inf)
    sels, ws = [], []
    wsum = jnp.zeros((TN, 1), jnp.float32)
    for k in range(3):
        m = jnp.min(d, axis=1, keepdims=True)
        cand = jnp.where(d == m, iota, jnp.float32(S))
        sel = jnp.min(cand, axis=1, keepdims=True)
        if k < 2:
            d = jnp.where(cand == sel, inf, d)
        wk = 1.0 / (m + 1e-8)
        wsum = wsum + wk
        sels.append(sel)
        ws.append(wk)
    inv = 1.0 / wsum
    base = jnp.float32(S) * b.astype(jnp.float32)
    t = jnp.concatenate(
        [sels[0] + base, sels[1] + base, sels[2] + base,
         ws[0] * inv, ws[1] * inv, ws[2] * inv,
         jnp.zeros((TN, 2), jnp.float32)], axis=1)
    tt = jnp.transpose(t, (1, 0))
    t8_ref[0] = tt
    i1_ref[...] = tt[0:1, :].astype(jnp.int32).reshape(TN)
    i2_ref[...] = tt[1:2, :].astype(jnp.int32).reshape(TN)
    i3_ref[...] = tt[2:3, :].astype(jnp.int32).reshape(TN)


def _make_sc_gather(BN, CH, NC):
    mesh = plsc.VectorSubcoreMesh(core_axis_name="c", subcore_axis_name="s")
    per_w = BN // 32

    @functools.partial(
        pl.kernel, mesh=mesh,
        compiler_params=pltpu.CompilerParams(use_tc_tiling_on_sc=False),
        out_type=jax.ShapeDtypeStruct((3 * BN, 64), jnp.float32),
        scratch_types=[
            pltpu.VMEM((CH,), jnp.int32),
            pltpu.VMEM((CH, 64), jnp.float32),
            pltpu.SemaphoreType.DMA,
        ],
    )
    def sc_gather(i1_hbm, i2_hbm, i3_hbm, table_hbm, out_hbm,
                  idx_v, rows_v, sem):
        wid = lax.axis_index("s") * NC + lax.axis_index("c")
        base = wid * per_w
        srcs = [i1_hbm, i2_hbm, i3_hbm]

        def chunk(c, _):
            qoff = base + c * CH
            for k in range(3):
                pltpu.sync_copy(srcs[k].at[pl.ds(qoff, CH)], idx_v)
                pltpu.async_copy(table_hbm.at[idx_v], rows_v, sem).wait()
                pltpu.sync_copy(rows_v, out_hbm.at[pl.ds(k * BN + qoff, CH)])
            return ()

        lax.fori_loop(0, per_w // CH, chunk, (), unroll=1)

    return sc_gather


def _p2_body(g1_ref, g2_ref, g3_ref, t8_ref, p1_ref, w0_ref, b0_ref,
             z0_ref, sum_ref, sq_ref):
    b = pl.program_id(0)
    nt = pl.program_id(1)

    @pl.when(jnp.logical_and(b == 0, nt == 0))
    def _():
        sum_ref[...] = jnp.zeros_like(sum_ref)
        sq_ref[...] = jnp.zeros_like(sq_ref)

    tq = jnp.transpose(t8_ref[0], (1, 0))
    interp_q = (tq[:, 3:4] * g1_ref[...] + tq[:, 4:5] * g2_ref[...]
                + tq[:, 5:6] * g3_ref[...])
    w0 = w0_ref[...]
    z0 = (_dot(w0[:, 0:64], p1_ref[0])
          + lax.dot_general(w0[:, 64:128], interp_q, (((1,), (1,)), ((), ())),
                            preferred_element_type=jnp.float32)
          + b0_ref[...])
    z0_ref[0] = z0
    sum_ref[...] += jnp.sum(z0, axis=1, keepdims=True)
    sq_ref[...] += jnp.sum(z0 * z0, axis=1, keepdims=True)


def _bn_affine(s_ref, q_ref, g_ref, be_ref, P):
    mean = s_ref[...] / P
    var = q_ref[...] / P - mean * mean
    sc = g_ref[...] * lax.rsqrt(var + 1e-5)
    sh = be_ref[...] - mean * sc
    return sc, sh


def _p3_body(z0_ref, s0_ref, q0_ref, g0_ref, be0_ref, w1_ref, b1_ref,
             z1_ref, sum_ref, sq_ref, *, P):
    b = pl.program_id(0)
    nt = pl.program_id(1)

    @pl.when(jnp.logical_and(b == 0, nt == 0))
    def _():
        sum_ref[...] = jnp.zeros_like(sum_ref)
        sq_ref[...] = jnp.zeros_like(sq_ref)

    sc, sh = _bn_affine(s0_ref, q0_ref, g0_ref, be0_ref, P)
    y0 = jnp.maximum(z0_ref[0] * sc + sh, 0.0)
    z1 = _dot(w1_ref[...], y0) + b1_ref[...]
    z1_ref[0] = z1
    sum_ref[...] += jnp.sum(z1, axis=1, keepdims=True)
    sq_ref[...] += jnp.sum(z1 * z1, axis=1, keepdims=True)


def _p4_body(z1_ref, s1_ref, q1_ref, g1_ref, be1_ref, out_ref, *, P):
    sc, sh = _bn_affine(s1_ref, q1_ref, g1_ref, be1_ref, P)
    out_ref[0] = jnp.maximum(z1_ref[0] * sc + sh, 0.0)


def _c64(i_map):
    return pl.BlockSpec((64, 1), i_map)


def kernel(xyz1, xyz2, points1, points2, W0, b0, g0, be0, W1, b1, g1, be1):
    B, _, N = xyz1.shape
    S = xyz2.shape[2]
    TN = 1024
    NT = N // TN
    P = float(B * N)
    BN = B * N
    CH = 128

    col = lambda v: v.reshape(64, 1)
    c0 = lambda b, n: (0, 0)

    t8, i1, i2, i3 = pl.pallas_call(
        functools.partial(_p1_body, TN=TN, S=S),
        grid=(B, NT),
        in_specs=[
            pl.BlockSpec((1, 3, TN), lambda b, n: (b, 0, n)),
            pl.BlockSpec((1, TN, 3), lambda b, n: (b, n, 0)),
            pl.BlockSpec((1, 3, S), lambda b, n: (b, 0, 0)),
        ],
        out_specs=[
            pl.BlockSpec((1, 8, TN), lambda b, n: (b, 0, n)),
            pl.BlockSpec((TN,), lambda b, n: (b * NT + n,)),
            pl.BlockSpec((TN,), lambda b, n: (b * NT + n,)),
            pl.BlockSpec((TN,), lambda b, n: (b * NT + n,)),
        ],
        out_shape=[jax.ShapeDtypeStruct((B, 8, N), jnp.float32),
                   jax.ShapeDtypeStruct((BN,), jnp.int32),
                   jax.ShapeDtypeStruct((BN,), jnp.int32),
                   jax.ShapeDtypeStruct((BN,), jnp.int32)],
    )(xyz1, jnp.transpose(xyz1, (0, 2, 1)), xyz2)

    p2_flat = jnp.transpose(points2, (0, 2, 1)).reshape(B * S, 64)
    g = _make_sc_gather(BN, CH, 2)(i1, i2, i3, p2_flat)

    f1 = jax.ShapeDtypeStruct((64, 1), jnp.float32)
    gspec = lambda k: pl.BlockSpec(
        (TN, 64), lambda b, n, k=k: (k * B * NT + b * NT + n, 0))
    z0, s0, q0 = pl.pallas_call(
        _p2_body,
        grid=(B, NT),
        in_specs=[
            gspec(0), gspec(1), gspec(2),
            pl.BlockSpec((1, 8, TN), lambda b, n: (b, 0, n)),
            pl.BlockSpec((1, 64, TN), lambda b, n: (b, 0, n)),
            pl.BlockSpec((64, 128), c0),
            _c64(c0),
        ],
        out_specs=[
            pl.BlockSpec((1, 64, TN), lambda b, n: (b, 0, n)),
            _c64(c0),
            _c64(c0),
        ],
        out_shape=[jax.ShapeDtypeStruct((B, 64, N), jnp.float32), f1, f1],
    )(g, g, g, t8, points1, W0, col(b0))

    z1, s1, q1 = pl.pallas_call(
        functools.partial(_p3_body, P=P),
        grid=(B, NT),
        in_specs=[
            pl.BlockSpec((1, 64, TN), lambda b, n: (b, 0, n)),
            _c64(c0), _c64(c0), _c64(c0), _c64(c0),
            pl.BlockSpec((64, 64), c0),
            _c64(c0),
        ],
        out_specs=[
            pl.BlockSpec((1, 64, TN), lambda b, n: (b, 0, n)),
            _c64(c0),
            _c64(c0),
        ],
        out_shape=[jax.ShapeDtypeStruct((B, 64, N), jnp.float32), f1, f1],
    )(z0, s0, q0, col(g0), col(be0), W1, col(b1))

    out = pl.pallas_call(
        functools.partial(_p4_body, P=P),
        grid=(B, NT),
        in_specs=[
            pl.BlockSpec((1, 64, TN), lambda b, n: (b, 0, n)),
            _c64(c0), _c64(c0), _c64(c0), _c64(c0),
        ],
        out_specs=pl.BlockSpec((1, 64, TN), lambda b, n: (b, 0, n)),
        out_shape=jax.ShapeDtypeStruct((B, 64, N), jnp.float32),
    )(z1, s1, q1, col(g1), col(be1))

    return out


# speedup vs baseline: 1.1386x; 1.1386x over previous
import functools

import jax
import jax.numpy as jnp
from jax import lax
from jax.experimental import pallas as pl
from jax.experimental.pallas import tpu as pltpu
from jax.experimental.pallas import tpu_sc as plsc


def _dot(a, b):
    return lax.dot_general(a, b, (((1,), (0,)), ((), ())),
                           preferred_element_type=jnp.float32)


def _p1_body(x1_ref, x1t_ref, x2_ref, t8_ref, i1_ref, i2_ref, i3_ref,
             *, TN, S):
    b = pl.program_id(0)

    x1b = x1_ref[0]
    x1t = x1t_ref[0]
    x2b = x2_ref[0]

    x2sq = x2b * x2b
    n2 = x2sq[0:1] + x2sq[1:2] + x2sq[2:3]
    x1tsq = x1t * x1t
    n1 = x1tsq[:, 0:1] + x1tsq[:, 1:2] + x1tsq[:, 2:3]
    prod = lax.dot_general(x1b, x2b, (((0,), (0,)), ((), ())),
                           preferred_element_type=jnp.float32)
    d = jnp.maximum(n1 + n2 - 2.0 * prod, 0.0)

    iota = lax.broadcasted_iota(jnp.int32, (TN, S), 1).astype(jnp.float32)
    inf = jnp.float32(jnp.inf)
    sels, ws = [], []
    wsum = jnp.zeros((TN, 1), jnp.float32)
    for k in range(3):
        m = jnp.min(d, axis=1, keepdims=True)
        cand = jnp.where(d == m, iota, jnp.float32(S))
        sel = jnp.min(cand, axis=1, keepdims=True)
        if k < 2:
            d = jnp.where(cand == sel, inf, d)
        wk = 1.0 / (m + 1e-8)
        wsum = wsum + wk
        sels.append(sel)
        ws.append(wk)
    inv = 1.0 / wsum
    base = jnp.float32(S) * b.astype(jnp.float32)
    t = jnp.concatenate(
        [sels[0] + base, sels[1] + base, sels[2] + base,
         ws[0] * inv, ws[1] * inv, ws[2] * inv,
         jnp.zeros((TN, 2), jnp.float32)], axis=1)
    tt = jnp.transpose(t, (1, 0))
    t8_ref[0] = tt
    i1_ref[...] = tt[0:1, :].astype(jnp.int32).reshape(TN)
    i2_ref[...] = tt[1:2, :].astype(jnp.int32).reshape(TN)
    i3_ref[...] = tt[2:3, :].astype(jnp.int32).reshape(TN)


def _make_sc_gather(BN, CH, NC):
    mesh = plsc.VectorSubcoreMesh(core_axis_name="c", subcore_axis_name="s")
    per_w = BN // 32

    @functools.partial(
        pl.kernel, mesh=mesh,
        compiler_params=pltpu.CompilerParams(use_tc_tiling_on_sc=False),
        out_type=jax.ShapeDtypeStruct((3 * BN, 128), jnp.float32),
        scratch_types=[
            pltpu.VMEM((CH,), jnp.int32),
            pltpu.VMEM((CH, 64), jnp.float32),
            pltpu.SemaphoreType.DMA,
        ],
    )
    def sc_gather(i1_hbm, i2_hbm, i3_hbm, table_hbm, out_hbm,
                  idx_v, rows_v, sem):
        wid = lax.axis_index("s") * NC + lax.axis_index("c")
        base = wid * per_w
        srcs = [i1_hbm, i2_hbm, i3_hbm]

        def chunk(c, _):
            qoff = base + c * CH
            for k in range(3):
                pltpu.sync_copy(srcs[k].at[pl.ds(qoff, CH)], idx_v)
                pltpu.async_copy(table_hbm.at[idx_v], rows_v, sem).wait()
                pltpu.sync_copy(
                    rows_v,
                    out_hbm.at[pl.ds(k * BN + qoff, CH), pl.ds(0, 64)])
            return ()

        lax.fori_loop(0, per_w // CH, chunk, (), unroll=1)

    return sc_gather


def _p2_body(g1_ref, g2_ref, g3_ref, t8_ref, p1_ref, w0_ref, b0_ref,
             z0_ref, sum_ref, sq_ref):
    b = pl.program_id(0)
    nt = pl.program_id(1)

    @pl.when(jnp.logical_and(b == 0, nt == 0))
    def _():
        sum_ref[...] = jnp.zeros_like(sum_ref)
        sq_ref[...] = jnp.zeros_like(sq_ref)

    tq = jnp.transpose(t8_ref[0], (1, 0))
    interp_q = (tq[:, 3:4] * g1_ref[:, 0:64] + tq[:, 4:5] * g2_ref[:, 0:64]
                + tq[:, 5:6] * g3_ref[:, 0:64])
    w0 = w0_ref[...]
    z0 = (_dot(w0[:, 0:64], p1_ref[0])
          + lax.dot_general(w0[:, 64:128], interp_q, (((1,), (1,)), ((), ())),
                            preferred_element_type=jnp.float32)
          + b0_ref[...])
    z0_ref[0] = z0
    sum_ref[...] += jnp.sum(z0, axis=1, keepdims=True)
    sq_ref[...] += jnp.sum(z0 * z0, axis=1, keepdims=True)


def _bn_affine(s_ref, q_ref, g_ref, be_ref, P):
    mean = s_ref[...] / P
    var = q_ref[...] / P - mean * mean
    sc = g_ref[...] * lax.rsqrt(var + 1e-5)
    sh = be_ref[...] - mean * sc
    return sc, sh


def _p3_body(z0_ref, s0_ref, q0_ref, g0_ref, be0_ref, w1_ref, b1_ref,
             z1_ref, sum_ref, sq_ref, *, P):
    b = pl.program_id(0)
    nt = pl.program_id(1)

    @pl.when(jnp.logical_and(b == 0, nt == 0))
    def _():
        sum_ref[...] = jnp.zeros_like(sum_ref)
        sq_ref[...] = jnp.zeros_like(sq_ref)

    sc, sh = _bn_affine(s0_ref, q0_ref, g0_ref, be0_ref, P)
    y0 = jnp.maximum(z0_ref[0] * sc + sh, 0.0)
    z1 = _dot(w1_ref[...], y0) + b1_ref[...]
    z1_ref[0] = z1
    sum_ref[...] += jnp.sum(z1, axis=1, keepdims=True)
    sq_ref[...] += jnp.sum(z1 * z1, axis=1, keepdims=True)


def _p4_body(z1_ref, s1_ref, q1_ref, g1_ref, be1_ref, out_ref, *, P):
    sc, sh = _bn_affine(s1_ref, q1_ref, g1_ref, be1_ref, P)
    out_ref[0] = jnp.maximum(z1_ref[0] * sc + sh, 0.0)


def _c64(i_map):
    return pl.BlockSpec((64, 1), i_map)


def kernel(xyz1, xyz2, points1, points2, W0, b0, g0, be0, W1, b1, g1, be1):
    B, _, N = xyz1.shape
    S = xyz2.shape[2]
    TN = 1024
    NT = N // TN
    P = float(B * N)
    BN = B * N
    CH = 128

    col = lambda v: v.reshape(64, 1)
    c0 = lambda b, n: (0, 0)

    t8, i1, i2, i3 = pl.pallas_call(
        functools.partial(_p1_body, TN=TN, S=S),
        grid=(B, NT),
        in_specs=[
            pl.BlockSpec((1, 3, TN), lambda b, n: (b, 0, n)),
            pl.BlockSpec((1, TN, 3), lambda b, n: (b, n, 0)),
            pl.BlockSpec((1, 3, S), lambda b, n: (b, 0, 0)),
        ],
        out_specs=[
            pl.BlockSpec((1, 8, TN), lambda b, n: (b, 0, n)),
            pl.BlockSpec((TN,), lambda b, n: (b * NT + n,)),
            pl.BlockSpec((TN,), lambda b, n: (b * NT + n,)),
            pl.BlockSpec((TN,), lambda b, n: (b * NT + n,)),
        ],
        out_shape=[jax.ShapeDtypeStruct((B, 8, N), jnp.float32),
                   jax.ShapeDtypeStruct((BN,), jnp.int32),
                   jax.ShapeDtypeStruct((BN,), jnp.int32),
                   jax.ShapeDtypeStruct((BN,), jnp.int32)],
    )(xyz1, jnp.transpose(xyz1, (0, 2, 1)), xyz2)

    p2_flat = jnp.transpose(points2, (0, 2, 1)).reshape(B * S, 64)
    g = _make_sc_gather(BN, CH, 2)(i1, i2, i3, p2_flat)

    f1 = jax.ShapeDtypeStruct((64, 1), jnp.float32)
    gspec = lambda k: pl.BlockSpec(
        (TN, 128), lambda b, n, k=k: (k * B * NT + b * NT + n, 0))
    z0, s0, q0 = pl.pallas_call(
        _p2_body,
        grid=(B, NT),
        in_specs=[
            gspec(0), gspec(1), gspec(2),
            pl.BlockSpec((1, 8, TN), lambda b, n: (b, 0, n)),
            pl.BlockSpec((1, 64, TN), lambda b, n: (b, 0, n)),
            pl.BlockSpec((64, 128), c0),
            _c64(c0),
        ],
        out_specs=[
            pl.BlockSpec((1, 64, TN), lambda b, n: (b, 0, n)),
            _c64(c0),
            _c64(c0),
        ],
        out_shape=[jax.ShapeDtypeStruct((B, 64, N), jnp.float32), f1, f1],
    )(g, g, g, t8, points1, W0, col(b0))

    z1, s1, q1 = pl.pallas_call(
        functools.partial(_p3_body, P=P),
        grid=(B, NT),
        in_specs=[
            pl.BlockSpec((1, 64, TN), lambda b, n: (b, 0, n)),
            _c64(c0), _c64(c0), _c64(c0), _c64(c0),
            pl.BlockSpec((64, 64), c0),
            _c64(c0),
        ],
        out_specs=[
            pl.BlockSpec((1, 64, TN), lambda b, n: (b, 0, n)),
            _c64(c0),
            _c64(c0),
        ],
        out_shape=[jax.ShapeDtypeStruct((B, 64, N), jnp.float32), f1, f1],
    )(z0, s0, q0, col(g0), col(be0), W1, col(b1))

    out = pl.pallas_call(
        functools.partial(_p4_body, P=P),
        grid=(B, NT),
        in_specs=[
            pl.BlockSpec((1, 64, TN), lambda b, n: (b, 0, n)),
            _c64(c0), _c64(c0), _c64(c0), _c64(c0),
        ],
        out_specs=pl.BlockSpec((1, 64, TN), lambda b, n: (b, 0, n)),
        out_shape=jax.ShapeDtypeStruct((B, 64, N), jnp.float32),
    )(z1, s1, q1, col(g1), col(be1))

    return out


# speedup vs baseline: 1.2089x; 1.0618x over previous
import functools

import jax
import jax.numpy as jnp
from jax import lax
from jax.experimental import pallas as pl
from jax.experimental.pallas import tpu as pltpu
from jax.experimental.pallas import tpu_sc as plsc


def _dot(a, b):
    return lax.dot_general(a, b, (((1,), (0,)), ((), ())),
                           preferred_element_type=jnp.float32)


def _p1_body(x1_ref, x1t_ref, x2_ref, t8_ref, i1_ref, i2_ref, i3_ref,
             *, TN, S):
    b = pl.program_id(0)

    x1b = x1_ref[0]
    x1t = x1t_ref[0]
    x2b = x2_ref[0]

    x2sq = x2b * x2b
    n2 = x2sq[0:1] + x2sq[1:2] + x2sq[2:3]
    x1tsq = x1t * x1t
    n1 = x1tsq[:, 0:1] + x1tsq[:, 1:2] + x1tsq[:, 2:3]
    prod = lax.dot_general(x1b, x2b, (((0,), (0,)), ((), ())),
                           preferred_element_type=jnp.float32)
    d = jnp.maximum(n1 + n2 - 2.0 * prod, 0.0)

    iota = lax.broadcasted_iota(jnp.int32, (TN, S), 1).astype(jnp.float32)
    inf = jnp.float32(jnp.inf)
    sels, ws = [], []
    wsum = jnp.zeros((TN, 1), jnp.float32)
    for k in range(3):
        m = jnp.min(d, axis=1, keepdims=True)
        cand = jnp.where(d == m, iota, jnp.float32(S))
        sel = jnp.min(cand, axis=1, keepdims=True)
        if k < 2:
            d = jnp.where(cand == sel, inf, d)
        wk = 1.0 / (m + 1e-8)
        wsum = wsum + wk
        sels.append(sel)
        ws.append(wk)
    inv = 1.0 / wsum
    base = jnp.float32(S) * b.astype(jnp.float32)
    t = jnp.concatenate(
        [sels[0] + base, sels[1] + base, sels[2] + base,
         ws[0] * inv, ws[1] * inv, ws[2] * inv,
         jnp.zeros((TN, 2), jnp.float32)], axis=1)
    tt = jnp.transpose(t, (1, 0))
    t8_ref[0] = tt
    i1_ref[...] = tt[0:1, :].astype(jnp.int32).reshape(TN)
    i2_ref[...] = tt[1:2, :].astype(jnp.int32).reshape(TN)
    i3_ref[...] = tt[2:3, :].astype(jnp.int32).reshape(TN)


def _make_sc_gather(BN, CH, NC):
    mesh = plsc.VectorSubcoreMesh(core_axis_name="c", subcore_axis_name="s")
    per_w = BN // 32

    NQC = per_w // CH

    @functools.partial(
        pl.kernel, mesh=mesh,
        compiler_params=pltpu.CompilerParams(use_tc_tiling_on_sc=False),
        out_type=jax.ShapeDtypeStruct((3 * BN, 128), jnp.float32),
        scratch_types=[
            pltpu.VMEM((per_w,), jnp.int32),
            pltpu.VMEM((per_w,), jnp.int32),
            pltpu.VMEM((per_w,), jnp.int32),
            pltpu.VMEM((CH, 64), jnp.float32),
            pltpu.VMEM((CH, 64), jnp.float32),
            pltpu.SemaphoreType.DMA,
            pltpu.SemaphoreType.DMA,
        ],
    )
    def sc_gather(i1_hbm, i2_hbm, i3_hbm, table_hbm, out_hbm,
                  i1_v, i2_v, i3_v, rows_a, rows_b, sem_a, sem_b):
        wid = lax.axis_index("s") * NC + lax.axis_index("c")
        base = wid * per_w
        pltpu.sync_copy(i1_hbm.at[pl.ds(base, per_w)], i1_v)
        pltpu.sync_copy(i2_hbm.at[pl.ds(base, per_w)], i2_v)
        pltpu.sync_copy(i3_hbm.at[pl.ds(base, per_w)], i3_v)
        idxs = (i1_v, i2_v, i3_v)
        bufs = (rows_a, rows_b)
        sems = (sem_a, sem_b)

        def gather_desc(k, qc, buf, sem):
            return pltpu.make_async_copy(
                table_hbm.at[idxs[k].at[pl.ds(qc * CH, CH)]], buf, sem)

        gather_desc(0, 0, bufs[0], sems[0]).start()

        def body(j, _):
            qc0 = 2 * j
            for t in range(6):
                k = t % 3
                qc = qc0 + t // 3
                cur = t % 2
                gather_desc(k, qc, bufs[cur], sems[cur]).wait()
                nk = (t + 1) % 3
                nqc = jnp.minimum(qc0 + (t + 1) // 3, NQC - 1)
                gather_desc(nk, nqc, bufs[(t + 1) % 2],
                            sems[(t + 1) % 2]).start()
                pltpu.sync_copy(
                    bufs[cur],
                    out_hbm.at[pl.ds(k * BN + base + qc * CH, CH),
                               pl.ds(0, 64)])
            return ()

        lax.fori_loop(0, NQC // 2, body, (), unroll=1)
        gather_desc(0, NQC - 1, bufs[0], sems[0]).wait()

    return sc_gather


def _p2_body(g1_ref, g2_ref, g3_ref, t8_ref, p1_ref, w0_ref, b0_ref,
             z0_ref, sum_ref, sq_ref):
    b = pl.program_id(0)
    nt = pl.program_id(1)

    @pl.when(jnp.logical_and(b == 0, nt == 0))
    def _():
        sum_ref[...] = jnp.zeros_like(sum_ref)
        sq_ref[...] = jnp.zeros_like(sq_ref)

    tq = jnp.transpose(t8_ref[0], (1, 0))
    interp_q = (tq[:, 3:4] * g1_ref[:, 0:64] + tq[:, 4:5] * g2_ref[:, 0:64]
                + tq[:, 5:6] * g3_ref[:, 0:64])
    w0 = w0_ref[...]
    z0 = (_dot(w0[:, 0:64], p1_ref[0])
          + lax.dot_general(w0[:, 64:128], interp_q, (((1,), (1,)), ((), ())),
                            preferred_element_type=jnp.float32)
          + b0_ref[...])
    z0_ref[0] = z0
    sum_ref[...] += jnp.sum(z0, axis=1, keepdims=True)
    sq_ref[...] += jnp.sum(z0 * z0, axis=1, keepdims=True)


def _bn_affine(s_ref, q_ref, g_ref, be_ref, P):
    mean = s_ref[...] / P
    var = q_ref[...] / P - mean * mean
    sc = g_ref[...] * lax.rsqrt(var + 1e-5)
    sh = be_ref[...] - mean * sc
    return sc, sh


def _p3_body(z0_ref, s0_ref, q0_ref, g0_ref, be0_ref, w1_ref, b1_ref,
             z1_ref, sum_ref, sq_ref, *, P):
    b = pl.program_id(0)
    nt = pl.program_id(1)

    @pl.when(jnp.logical_and(b == 0, nt == 0))
    def _():
        sum_ref[...] = jnp.zeros_like(sum_ref)
        sq_ref[...] = jnp.zeros_like(sq_ref)

    sc, sh = _bn_affine(s0_ref, q0_ref, g0_ref, be0_ref, P)
    y0 = jnp.maximum(z0_ref[0] * sc + sh, 0.0)
    z1 = _dot(w1_ref[...], y0) + b1_ref[...]
    z1_ref[0] = z1
    sum_ref[...] += jnp.sum(z1, axis=1, keepdims=True)
    sq_ref[...] += jnp.sum(z1 * z1, axis=1, keepdims=True)


def _p4_body(z1_ref, s1_ref, q1_ref, g1_ref, be1_ref, out_ref, *, P):
    sc, sh = _bn_affine(s1_ref, q1_ref, g1_ref, be1_ref, P)
    out_ref[0] = jnp.maximum(z1_ref[0] * sc + sh, 0.0)


def _c64(i_map):
    return pl.BlockSpec((64, 1), i_map)


def kernel(xyz1, xyz2, points1, points2, W0, b0, g0, be0, W1, b1, g1, be1):
    B, _, N = xyz1.shape
    S = xyz2.shape[2]
    TN = 1024
    NT = N // TN
    P = float(B * N)
    BN = B * N
    CH = 128

    col = lambda v: v.reshape(64, 1)
    c0 = lambda b, n: (0, 0)

    t8, i1, i2, i3 = pl.pallas_call(
        functools.partial(_p1_body, TN=TN, S=S),
        grid=(B, NT),
        in_specs=[
            pl.BlockSpec((1, 3, TN), lambda b, n: (b, 0, n)),
            pl.BlockSpec((1, TN, 3), lambda b, n: (b, n, 0)),
            pl.BlockSpec((1, 3, S), lambda b, n: (b, 0, 0)),
        ],
        out_specs=[
            pl.BlockSpec((1, 8, TN), lambda b, n: (b, 0, n)),
            pl.BlockSpec((TN,), lambda b, n: (b * NT + n,)),
            pl.BlockSpec((TN,), lambda b, n: (b * NT + n,)),
            pl.BlockSpec((TN,), lambda b, n: (b * NT + n,)),
        ],
        out_shape=[jax.ShapeDtypeStruct((B, 8, N), jnp.float32),
                   jax.ShapeDtypeStruct((BN,), jnp.int32),
                   jax.ShapeDtypeStruct((BN,), jnp.int32),
                   jax.ShapeDtypeStruct((BN,), jnp.int32)],
    )(xyz1, jnp.transpose(xyz1, (0, 2, 1)), xyz2)

    p2_flat = jnp.transpose(points2, (0, 2, 1)).reshape(B * S, 64)
    g = _make_sc_gather(BN, CH, 2)(i1, i2, i3, p2_flat)

    f1 = jax.ShapeDtypeStruct((64, 1), jnp.float32)
    gspec = lambda k: pl.BlockSpec(
        (TN, 128), lambda b, n, k=k: (k * B * NT + b * NT + n, 0))
    z0, s0, q0 = pl.pallas_call(
        _p2_body,
        grid=(B, NT),
        in_specs=[
            gspec(0), gspec(1), gspec(2),
            pl.BlockSpec((1, 8, TN), lambda b, n: (b, 0, n)),
            pl.BlockSpec((1, 64, TN), lambda b, n: (b, 0, n)),
            pl.BlockSpec((64, 128), c0),
            _c64(c0),
        ],
        out_specs=[
            pl.BlockSpec((1, 64, TN), lambda b, n: (b, 0, n)),
            _c64(c0),
            _c64(c0),
        ],
        out_shape=[jax.ShapeDtypeStruct((B, 64, N), jnp.float32), f1, f1],
    )(g, g, g, t8, points1, W0, col(b0))

    z1, s1, q1 = pl.pallas_call(
        functools.partial(_p3_body, P=P),
        grid=(B, NT),
        in_specs=[
            pl.BlockSpec((1, 64, TN), lambda b, n: (b, 0, n)),
            _c64(c0), _c64(c0), _c64(c0), _c64(c0),
            pl.BlockSpec((64, 64), c0),
            _c64(c0),
        ],
        out_specs=[
            pl.BlockSpec((1, 64, TN), lambda b, n: (b, 0, n)),
            _c64(c0),
            _c64(c0),
        ],
        out_shape=[jax.ShapeDtypeStruct((B, 64, N), jnp.float32), f1, f1],
    )(z0, s0, q0, col(g0), col(be0), W1, col(b1))

    out = pl.pallas_call(
        functools.partial(_p4_body, P=P),
        grid=(B, NT),
        in_specs=[
            pl.BlockSpec((1, 64, TN), lambda b, n: (b, 0, n)),
            _c64(c0), _c64(c0), _c64(c0), _c64(c0),
        ],
        out_specs=pl.BlockSpec((1, 64, TN), lambda b, n: (b, 0, n)),
        out_shape=jax.ShapeDtypeStruct((B, 64, N), jnp.float32),
    )(z1, s1, q1, col(g1), col(be1))

    return out
